# Initial kernel scaffold; baseline (speedup 1.0000x reference)
import jax
import jax.numpy as jnp
from jax.experimental import pallas as pl


def kernel(x, edge_index, batch, W1, b1, W2, b2, W3, b3, g1, be1, g2, be2, g3, be3, cW1, cb1, cW2, cb2):
    raise NotImplementedError("write your pallas kernel here")


# speedup vs baseline: 12.9396x; 12.9396x over previous
import functools

import jax
import jax.numpy as jnp
from jax import lax
from jax.experimental import pallas as pl
from jax.experimental.pallas import tpu as pltpu
from jax.experimental.pallas import tpu_sc as plsc

N_NODES = 10000
N_EDGES = 320000
D = 128
H = 128
NUM_CLASSES = 10
NUM_GRAPHS = 128

_NC = 2
_NS = 16
_CH = 128
_CHUNKS = N_EDGES // _CH
_CHUNKS_PER_CORE = _CHUNKS // _NC
_ROWS_PER_TILE = 632
_N_PAD = _ROWS_PER_TILE * _NS
_DEG_PAD = 640 * _NS


def _sc_mesh():
    return plsc.VectorSubcoreMesh(core_axis_name="c", subcore_axis_name="s")


def _sc_degree(dst, zdeg):
    @functools.partial(
        pl.kernel,
        mesh=_sc_mesh(),
        out_type=jax.ShapeDtypeStruct((_NC, _DEG_PAD), jnp.float32),
        scratch_types=[
            pltpu.VMEM((_CH,), jnp.int32),
            pltpu.VMEM((_CH,), jnp.float32),
            pltpu.VMEM_SHARED((_DEG_PAD,), jnp.float32),
        ],
    )
    def k(dst_hbm, z_hbm, out_hbm, dst_v, ones_v, acc_sh):
        c = lax.axis_index("c")
        s = lax.axis_index("s")
        e0 = s * 640
        pltpu.sync_copy(z_hbm.at[pl.ds(e0, 640)], acc_sh.at[pl.ds(e0, 640)])
        for j in range(_CH // 16):
            ones_v[pl.ds(j * 16, 16)] = jnp.ones((16,), jnp.float32)
        plsc.subcore_barrier()

        n_iter = (_CHUNKS_PER_CORE - s + _NS - 1) // _NS

        def body(i, carry):
            kk = s + i * _NS
            base = (c * _CHUNKS_PER_CORE + kk) * _CH
            pltpu.sync_copy(dst_hbm.at[pl.ds(base, _CH)], dst_v)
            pltpu.sync_copy(ones_v, acc_sh.at[dst_v], add=True)
            return carry

        lax.fori_loop(0, n_iter, body, 0)
        plsc.subcore_barrier()
        pltpu.sync_copy(acc_sh.at[pl.ds(e0, 640)], out_hbm.at[c, pl.ds(e0, 640)])

    return k(dst, zdeg)


def _sc_scatter_rows(y, src, dst, zrows):
    @functools.partial(
        pl.kernel,
        mesh=_sc_mesh(),
        out_type=jax.ShapeDtypeStruct((_NC, _N_PAD, H), jnp.float32),
        scratch_types=[
            pltpu.VMEM((_CH,), jnp.int32),
            pltpu.VMEM((_CH,), jnp.int32),
            pltpu.VMEM((_CH, H), jnp.float32),
            pltpu.VMEM_SHARED((_N_PAD, H), jnp.float32),
            pltpu.SemaphoreType.DMA,
        ],
    )
    def k(y_hbm, src_hbm, dst_hbm, z_hbm, out_hbm, src_v, dst_v, rows_v, acc_sh, sem):
        c = lax.axis_index("c")
        s = lax.axis_index("s")
        r0 = s * _ROWS_PER_TILE
        pltpu.sync_copy(z_hbm.at[pl.ds(r0, _ROWS_PER_TILE)],
                        acc_sh.at[pl.ds(r0, _ROWS_PER_TILE)])
        plsc.subcore_barrier()

        n_iter = (_CHUNKS_PER_CORE - s + _NS - 1) // _NS

        def body(i, carry):
            kk = s + i * _NS
            base = (c * _CHUNKS_PER_CORE + kk) * _CH
            pltpu.sync_copy(src_hbm.at[pl.ds(base, _CH)], src_v)
            pltpu.sync_copy(dst_hbm.at[pl.ds(base, _CH)], dst_v)
            pltpu.async_copy(y_hbm.at[src_v], rows_v, sem).wait()
            pltpu.sync_copy(rows_v, acc_sh.at[dst_v], add=True)
            return carry

        lax.fori_loop(0, n_iter, body, 0)
        plsc.subcore_barrier()
        pltpu.sync_copy(acc_sh.at[pl.ds(r0, _ROWS_PER_TILE)],
                        out_hbm.at[c, pl.ds(r0, _ROWS_PER_TILE)])

    return k(y, src, dst, zrows)


def _dinv_of(degt):
    deg = degt[:, 0:1] + degt[:, 1:2] + 1.0
    return lax.rsqrt(jnp.maximum(deg, 1e-12))


def _tc_pre_body(x_ref, w_ref, degt_ref, y_ref):
    dinv = _dinv_of(degt_ref[...])
    y_ref[...] = jnp.dot(x_ref[...], w_ref[...],
                         preferred_element_type=jnp.float32) * dinv


def _tc_pre(x, W1, degt):
    return pl.pallas_call(
        _tc_pre_body,
        out_shape=jax.ShapeDtypeStruct((N_NODES, H), jnp.float32),
    )(x, W1, degt)


def _tc_mid_body(sa_ref, sb_ref, y_ref, degt_ref, b_ref, g_ref, be_ref, w_ref, o_ref):
    dinv = _dinv_of(degt_ref[...])
    t = dinv * (sa_ref[...] + sb_ref[...] + y_ref[...]) + b_ref[...]
    mu = jnp.mean(t, axis=0, keepdims=True)
    var = jnp.mean((t - mu) ** 2, axis=0, keepdims=True)
    h = (t - mu) * lax.rsqrt(var + 1e-5) * g_ref[...] + be_ref[...]
    h = jnp.maximum(h, 0.0)
    o_ref[...] = jnp.dot(h, w_ref[...], preferred_element_type=jnp.float32) * dinv


def _tc_mid(sa, sb, y, degt, b, g, be, Wn):
    return pl.pallas_call(
        _tc_mid_body,
        out_shape=jax.ShapeDtypeStruct((N_NODES, H), jnp.float32),
    )(sa, sb, y, degt, b, g, be, Wn)


def _tc_post_body(sa_ref, sb_ref, y_ref, degt_ref, b_ref, g_ref, be_ref,
                  x_ref, batch_ref, cw1_ref, cb1_ref, cw2_ref, cb2_ref, o_ref):
    dinv = _dinv_of(degt_ref[...])
    t = dinv * (sa_ref[...] + sb_ref[...] + y_ref[...]) + b_ref[...]
    mu = jnp.mean(t, axis=0, keepdims=True)
    var = jnp.mean((t - mu) ** 2, axis=0, keepdims=True)
    h = (t - mu) * lax.rsqrt(var + 1e-5) * g_ref[...] + be_ref[...]

    gids = lax.broadcasted_iota(jnp.int32, (NUM_GRAPHS, N_NODES), 0)
    onehot = (gids == batch_ref[...]).astype(jnp.float32)
    counts = jnp.sum(onehot, axis=1, keepdims=True)
    inv_cnt = 1.0 / jnp.maximum(counts, 1.0)
    ph = jnp.dot(onehot, h, preferred_element_type=jnp.float32) * inv_cnt
    px = jnp.dot(onehot, x_ref[...], preferred_element_type=jnp.float32) * inv_cnt
    comb = jnp.concatenate([ph, px], axis=1)
    z = jnp.maximum(jnp.dot(comb, cw1_ref[...],
                            preferred_element_type=jnp.float32) + cb1_ref[...], 0.0)
    o_ref[...] = jnp.dot(z, cw2_ref[...],
                         preferred_element_type=jnp.float32) + cb2_ref[...]


def _tc_post(sa, sb, y, degt, b, g, be, x, batch2d, cW1, cb1, cW2, cb2):
    return pl.pallas_call(
        _tc_post_body,
        out_shape=jax.ShapeDtypeStruct((NUM_GRAPHS, NUM_CLASSES), jnp.float32),
    )(sa, sb, y, degt, b, g, be, x, batch2d, cW1, cb1, cW2, cb2)


def kernel(x, edge_index, batch, W1, b1, W2, b2, W3, b3,
           g1, be1, g2, be2, g3, be3, cW1, cb1, cW2, cb2):
    src = edge_index[0].astype(jnp.int32)
    dst = edge_index[1].astype(jnp.int32)
    batch2d = batch.astype(jnp.int32).reshape(1, N_NODES)

    zdeg = jnp.zeros((_DEG_PAD,), jnp.float32)
    zrows = jnp.zeros((_N_PAD, H), jnp.float32)

    degp = _sc_degree(dst, zdeg)
    degt = degp[:, :N_NODES].T

    b1r, b2r, b3r = b1.reshape(1, H), b2.reshape(1, H), b3.reshape(1, H)
    g1r, g2r, g3r = g1.reshape(1, H), g2.reshape(1, H), g3.reshape(1, H)
    be1r, be2r, be3r = be1.reshape(1, H), be2.reshape(1, H), be3.reshape(1, H)
    cb1r, cb2r = cb1.reshape(1, H // 2), cb2.reshape(1, NUM_CLASSES)

    y1 = _tc_pre(x, W1, degt)
    s1 = _sc_scatter_rows(y1, src, dst, zrows)[:, :N_NODES]
    y2 = _tc_mid(s1[0], s1[1], y1, degt, b1r, g1r, be1r, W2)
    s2 = _sc_scatter_rows(y2, src, dst, zrows)[:, :N_NODES]
    y3 = _tc_mid(s2[0], s2[1], y2, degt, b2r, g2r, be2r, W3)
    s3 = _sc_scatter_rows(y3, src, dst, zrows)[:, :N_NODES]
    return _tc_post(s3[0], s3[1], y3, degt, b3r, g3r, be3r,
                    x, batch2d, cW1, cb1r, cW2, cb2r)


# speedup vs baseline: 21.4323x; 1.6563x over previous
import functools

import jax
import jax.numpy as jnp
from jax import lax
from jax.experimental import pallas as pl
from jax.experimental.pallas import tpu as pltpu
from jax.experimental.pallas import tpu_sc as plsc

N_NODES = 10000
N_EDGES = 320000
D = 128
H = 128
NUM_CLASSES = 10
NUM_GRAPHS = 128

_NC = 2
_NS = 16
_CH = 128
_CHUNKS = N_EDGES // _CH
_CHUNKS_PER_CORE = _CHUNKS // _NC
_ROWS_PER_TILE = 632
_N_PAD = _ROWS_PER_TILE * _NS
_DEG_PAD = 640 * _NS


def _sc_mesh():
    return plsc.VectorSubcoreMesh(core_axis_name="c", subcore_axis_name="s")


def _sc_degree(dst, zdeg):
    nbd = 4

    @functools.partial(
        pl.kernel,
        mesh=_sc_mesh(),
        out_type=jax.ShapeDtypeStruct((_NC, _DEG_PAD), jnp.float32),
        scratch_types=[
            pltpu.VMEM((nbd, _CH), jnp.int32),
            pltpu.VMEM((_CH,), jnp.float32),
            pltpu.VMEM_SHARED((_DEG_PAD,), jnp.float32),
        ] + [pltpu.SemaphoreType.DMA] * nbd,
    )
    def k(dst_hbm, z_hbm, out_hbm, dst_v, ones_v, acc_sh, *sems):
        c = lax.axis_index("c")
        s = lax.axis_index("s")
        e0 = s * 640
        pltpu.sync_copy(z_hbm.at[pl.ds(e0, 640)], acc_sh.at[pl.ds(e0, 640)])
        for j in range(_CH // 16):
            ones_v[pl.ds(j * 16, 16)] = jnp.ones((16,), jnp.float32)

        n_iter = (_CHUNKS_PER_CORE - s + _NS - 1) // _NS

        def idx_copy(i, b):
            base = (c * _CHUNKS_PER_CORE + s + i * _NS) * _CH
            return pltpu.make_async_copy(dst_hbm.at[pl.ds(base, _CH)],
                                         dst_v.at[b], sems[b])

        for b in range(nbd):
            @pl.when(b < n_iter)
            def _(b=b):
                idx_copy(b, b).start()

        plsc.subcore_barrier()

        def group(g, carry):
            for b in range(nbd):
                i = g * nbd + b

                @pl.when(i < n_iter)
                def _(i=i, b=b):
                    idx_copy(i, b).wait()
                    pltpu.sync_copy(ones_v, acc_sh.at[dst_v.at[b]], add=True)

                    @pl.when(i + nbd < n_iter)
                    def _(i=i, b=b):
                        idx_copy(i + nbd, b).start()
            return carry

        lax.fori_loop(0, (n_iter + nbd - 1) // nbd, group, 0)
        plsc.subcore_barrier()
        pltpu.sync_copy(acc_sh.at[pl.ds(e0, 640)], out_hbm.at[c, pl.ds(e0, 640)])

    return k(dst, zdeg)


_NB = 3


def _sc_scatter_rows(y, src, dst, zrows):
    @functools.partial(
        pl.kernel,
        mesh=_sc_mesh(),
        out_type=jax.ShapeDtypeStruct((_NC, _N_PAD, H), jnp.float32),
        scratch_types=[
            pltpu.VMEM((_NB, _CH), jnp.int32),
            pltpu.VMEM((_NB, _CH), jnp.int32),
            pltpu.VMEM((_NB, _CH, H), jnp.float32),
            pltpu.VMEM_SHARED((_N_PAD, H), jnp.float32),
        ] + [pltpu.SemaphoreType.DMA] * (2 * _NB),
    )
    def k(y_hbm, src_hbm, dst_hbm, z_hbm, out_hbm, src_v, dst_v, rows_v, acc_sh, *sems):
        sem_i = sems[:_NB]
        sem_g = sems[_NB:]
        c = lax.axis_index("c")
        s = lax.axis_index("s")
        r0 = s * _ROWS_PER_TILE
        pltpu.sync_copy(z_hbm.at[pl.ds(r0, _ROWS_PER_TILE)],
                        acc_sh.at[pl.ds(r0, _ROWS_PER_TILE)])

        n_iter = (_CHUNKS_PER_CORE - s + _NS - 1) // _NS

        def ebase(i):
            return (c * _CHUNKS_PER_CORE + s + i * _NS) * _CH

        def idx_copies(i, b):
            return (pltpu.make_async_copy(src_hbm.at[pl.ds(ebase(i), _CH)],
                                          src_v.at[b], sem_i[b]),
                    pltpu.make_async_copy(dst_hbm.at[pl.ds(ebase(i), _CH)],
                                          dst_v.at[b], sem_i[b]))

        def gather_copy(b):
            return pltpu.make_async_copy(y_hbm.at[src_v.at[b]], rows_v.at[b],
                                         sem_g[b])

        for b in range(_NB):
            @pl.when(b < n_iter)
            def _(b=b):
                for cp in idx_copies(b, b):
                    cp.start()

        plsc.subcore_barrier()

        def group(g, carry):
            for b in range(_NB):
                i = g * _NB + b

                @pl.when(i < n_iter)
                def _(i=i, b=b):
                    for cp in idx_copies(i, b):
                        cp.wait()
                    gather_copy(b).start()
            for b in range(_NB):
                i = g * _NB + b

                @pl.when(i < n_iter)
                def _(i=i, b=b):
                    gather_copy(b).wait()
                    pltpu.sync_copy(rows_v.at[b], acc_sh.at[dst_v.at[b]], add=True)

                    @pl.when(i + _NB < n_iter)
                    def _(i=i, b=b):
                        for cp in idx_copies(i + _NB, b):
                            cp.start()
            return carry

        lax.fori_loop(0, (n_iter + _NB - 1) // _NB, group, 0)
        plsc.subcore_barrier()
        pltpu.sync_copy(acc_sh.at[pl.ds(r0, _ROWS_PER_TILE)],
                        out_hbm.at[c, pl.ds(r0, _ROWS_PER_TILE)])

    return k(y, src, dst, zrows)


def _dinv_of(degt):
    deg = degt[:, 0:1] + degt[:, 1:2] + 1.0
    return lax.rsqrt(jnp.maximum(deg, 1e-12))


def _tc_pre_body(x_ref, w_ref, degt_ref, y_ref):
    dinv = _dinv_of(degt_ref[...])
    y_ref[...] = jnp.dot(x_ref[...], w_ref[...],
                         preferred_element_type=jnp.float32) * dinv


def _tc_pre(x, W1, degt):
    return pl.pallas_call(
        _tc_pre_body,
        out_shape=jax.ShapeDtypeStruct((N_NODES, H), jnp.float32),
    )(x, W1, degt)


def _tc_mid_body(sa_ref, sb_ref, y_ref, degt_ref, b_ref, g_ref, be_ref, w_ref, o_ref):
    dinv = _dinv_of(degt_ref[...])
    t = dinv * (sa_ref[...] + sb_ref[...] + y_ref[...]) + b_ref[...]
    mu = jnp.mean(t, axis=0, keepdims=True)
    var = jnp.mean((t - mu) ** 2, axis=0, keepdims=True)
    h = (t - mu) * lax.rsqrt(var + 1e-5) * g_ref[...] + be_ref[...]
    h = jnp.maximum(h, 0.0)
    o_ref[...] = jnp.dot(h, w_ref[...], preferred_element_type=jnp.float32) * dinv


def _tc_mid(sa, sb, y, degt, b, g, be, Wn):
    return pl.pallas_call(
        _tc_mid_body,
        out_shape=jax.ShapeDtypeStruct((N_NODES, H), jnp.float32),
    )(sa, sb, y, degt, b, g, be, Wn)


def _tc_post_body(sa_ref, sb_ref, y_ref, degt_ref, b_ref, g_ref, be_ref,
                  x_ref, batch_ref, cw1_ref, cb1_ref, cw2_ref, cb2_ref, o_ref):
    dinv = _dinv_of(degt_ref[...])
    t = dinv * (sa_ref[...] + sb_ref[...] + y_ref[...]) + b_ref[...]
    mu = jnp.mean(t, axis=0, keepdims=True)
    var = jnp.mean((t - mu) ** 2, axis=0, keepdims=True)
    h = (t - mu) * lax.rsqrt(var + 1e-5) * g_ref[...] + be_ref[...]

    gids = lax.broadcasted_iota(jnp.int32, (NUM_GRAPHS, N_NODES), 0)
    onehot = (gids == batch_ref[...]).astype(jnp.float32)
    counts = jnp.sum(onehot, axis=1, keepdims=True)
    inv_cnt = 1.0 / jnp.maximum(counts, 1.0)
    ph = jnp.dot(onehot, h, preferred_element_type=jnp.float32) * inv_cnt
    px = jnp.dot(onehot, x_ref[...], preferred_element_type=jnp.float32) * inv_cnt
    comb = jnp.concatenate([ph, px], axis=1)
    z = jnp.maximum(jnp.dot(comb, cw1_ref[...],
                            preferred_element_type=jnp.float32) + cb1_ref[...], 0.0)
    o_ref[...] = jnp.dot(z, cw2_ref[...],
                         preferred_element_type=jnp.float32) + cb2_ref[...]


def _tc_post(sa, sb, y, degt, b, g, be, x, batch2d, cW1, cb1, cW2, cb2):
    return pl.pallas_call(
        _tc_post_body,
        out_shape=jax.ShapeDtypeStruct((NUM_GRAPHS, NUM_CLASSES), jnp.float32),
    )(sa, sb, y, degt, b, g, be, x, batch2d, cW1, cb1, cW2, cb2)


def kernel(x, edge_index, batch, W1, b1, W2, b2, W3, b3,
           g1, be1, g2, be2, g3, be3, cW1, cb1, cW2, cb2):
    src = edge_index[0].astype(jnp.int32)
    dst = edge_index[1].astype(jnp.int32)
    batch2d = batch.astype(jnp.int32).reshape(1, N_NODES)

    zdeg = jnp.zeros((_DEG_PAD,), jnp.float32)
    zrows = jnp.zeros((_N_PAD, H), jnp.float32)

    degp = _sc_degree(dst, zdeg)
    degt = degp[:, :N_NODES].T

    b1r, b2r, b3r = b1.reshape(1, H), b2.reshape(1, H), b3.reshape(1, H)
    g1r, g2r, g3r = g1.reshape(1, H), g2.reshape(1, H), g3.reshape(1, H)
    be1r, be2r, be3r = be1.reshape(1, H), be2.reshape(1, H), be3.reshape(1, H)
    cb1r, cb2r = cb1.reshape(1, H // 2), cb2.reshape(1, NUM_CLASSES)

    y1 = _tc_pre(x, W1, degt)
    s1 = _sc_scatter_rows(y1, src, dst, zrows)[:, :N_NODES]
    y2 = _tc_mid(s1[0], s1[1], y1, degt, b1r, g1r, be1r, W2)
    s2 = _sc_scatter_rows(y2, src, dst, zrows)[:, :N_NODES]
    y3 = _tc_mid(s2[0], s2[1], y2, degt, b2r, g2r, be2r, W3)
    s3 = _sc_scatter_rows(y3, src, dst, zrows)[:, :N_NODES]
    return _tc_post(s3[0], s3[1], y3, degt, b3r, g3r, be3r,
                    x, batch2d, cW1, cb1r, cW2, cb2r)
